# Initial kernel scaffold; baseline (speedup 1.0000x reference)
import jax
import jax.numpy as jnp
from jax.experimental import pallas as pl


def kernel(x, edge_index, atomic_numbers, coeffs):
    raise NotImplementedError("write your pallas kernel here")


# speedup vs baseline: 2.0784x; 2.0784x over previous
import functools
import math

import numpy as np
import jax
import jax.numpy as jnp
from jax import lax
from jax.experimental import pallas as pl
from jax.experimental.pallas import tpu as pltpu
from jax.experimental.pallas import tpu_sc as plsc

_L = 2
_N_NODES = 10000
_N_EDGES = 160000
_MAX_NUMBER = 8
_R_CUTOFF = 6.0
_N_RS = 64
_ALPHA = 16.0

def _gen_l_list():
    rows = []
    for single_l in range(_L + 1):
        for ii in range(single_l + 1):
            for jj in range(single_l + 1):
                for kk in range(single_l + 1):
                    if ii + jj + kk == single_l:
                        rows.append([ii, jj, kk])
    return np.array(rows, dtype=np.int32)

_L_LIST = _gen_l_list()
_N_L = _L_LIST.shape[0]
_L_SUM = _L_LIST.sum(-1)
_fact = np.vectorize(math.factorial)
_L_PREIDX = (_fact(_L_SUM.astype(np.int64)) /
             _fact(_L_LIST.astype(np.int64)).prod(-1)).astype(np.float32)
_R_S = np.linspace(0.0, 6.0, _N_RS).astype(np.float32)

_PREIDX_ROW = np.tile(_L_PREIDX, _N_RS)[None, :]
_W_PROJ = np.zeros((_N_RS * _N_L, _N_RS * (_L + 1)), dtype=np.float32)
for _k in range(_N_RS):
    for _l in range(_N_L):
        _W_PROJ[_k * _N_L + _l, _k * (_L + 1) + _L_SUM[_l]] = 1.0

_E_PAD = 163840
_EB = 1280
_N_EB = _E_PAD // _EB
_NC = 2000
_N_NB = _N_NODES // _NC
_NW = 32
_EPW = _E_PAD // _NW


def _sc_edge_kernel(xx_hbm, xy_hbm, xz_hbm, an_hbm, cf_hbm, src_hbm, dst_hbm,
                    o_rx, o_ry, o_rz, o_r2, o_cf,
                    xx_v, xy_v, xz_v, an_v, cf_v, src_v, dst_v,
                    rx_v, ry_v, rz_v, r2_v, cfo_v):
    wid = lax.axis_index("s") * 2 + lax.axis_index("c")
    base = wid * _EPW
    pltpu.sync_copy(xx_hbm, xx_v)
    pltpu.sync_copy(xy_hbm, xy_v)
    pltpu.sync_copy(xz_hbm, xz_v)
    pltpu.sync_copy(an_hbm, an_v)
    pltpu.sync_copy(cf_hbm, cf_v)
    pltpu.sync_copy(src_hbm.at[pl.ds(base, _EPW)], src_v)
    pltpu.sync_copy(dst_hbm.at[pl.ds(base, _EPW)], dst_v)

    def body(i, _):
        off = i * 16
        s_idx = src_v[pl.ds(off, 16)]
        d_idx = dst_v[pl.ds(off, 16)]
        xs = plsc.load_gather(xx_v, [s_idx])
        ys = plsc.load_gather(xy_v, [s_idx])
        zs = plsc.load_gather(xz_v, [s_idx])
        xd = plsc.load_gather(xx_v, [d_idx])
        yd = plsc.load_gather(xy_v, [d_idx])
        zd = plsc.load_gather(xz_v, [d_idx])
        rx = (xd - xs) + 1e-6
        ry = (yd - ys) + 1e-6
        rz = (zd - zs) + 1e-6
        r2 = rx * rx + ry * ry + rz * rz
        zsrc = plsc.load_gather(an_v, [s_idx])
        zdst = plsc.load_gather(an_v, [d_idx])
        zmin = jnp.minimum(zsrc, zdst)
        zmax = jnp.maximum(zsrc, zdst)
        cf = plsc.load_gather(cf_v, [zmin * _MAX_NUMBER + zmax])
        rx_v[pl.ds(off, 16)] = rx
        ry_v[pl.ds(off, 16)] = ry
        rz_v[pl.ds(off, 16)] = rz
        r2_v[pl.ds(off, 16)] = r2
        cfo_v[pl.ds(off, 16)] = cf
        return 0

    lax.fori_loop(0, _EPW // 16, body, 0)

    pltpu.sync_copy(rx_v, o_rx.at[pl.ds(base, _EPW)])
    pltpu.sync_copy(ry_v, o_ry.at[pl.ds(base, _EPW)])
    pltpu.sync_copy(rz_v, o_rz.at[pl.ds(base, _EPW)])
    pltpu.sync_copy(r2_v, o_r2.at[pl.ds(base, _EPW)])
    pltpu.sync_copy(cfo_v, o_cf.at[pl.ds(base, _EPW)])


def _make_sc_edge():
    return functools.partial(
        pl.kernel,
        out_type=[jax.ShapeDtypeStruct((_E_PAD,), jnp.float32)] * 5,
        mesh=plsc.VectorSubcoreMesh(core_axis_name="c", subcore_axis_name="s"),
        compiler_params=pltpu.CompilerParams(needs_layout_passes=False),
        scratch_types=[
        pltpu.VMEM((_N_NODES,), jnp.float32),
        pltpu.VMEM((_N_NODES,), jnp.float32),
        pltpu.VMEM((_N_NODES,), jnp.float32),
        pltpu.VMEM((_N_NODES,), jnp.int32),
        pltpu.VMEM((_MAX_NUMBER * _MAX_NUMBER,), jnp.float32),
        pltpu.VMEM((_EPW,), jnp.int32),
        pltpu.VMEM((_EPW,), jnp.int32),
        pltpu.VMEM((_EPW,), jnp.float32),
        pltpu.VMEM((_EPW,), jnp.float32),
        pltpu.VMEM((_EPW,), jnp.float32),
            pltpu.VMEM((_EPW,), jnp.float32),
            pltpu.VMEM((_EPW,), jnp.float32),
        ],
    )(_sc_edge_kernel)


def _tc_dense_kernel(rx_ref, ry_ref, rz_ref, r2_ref, cf_ref, dst_ref,
                     rs_ref, preidx_ref, w_ref, out_ref, acc_ref):
    n = pl.program_id(0)
    e = pl.program_id(1)

    @pl.when(e == 0)
    def _():
        acc_ref[...] = jnp.zeros_like(acc_ref)

    rx = rx_ref[0]
    ry = ry_ref[0]
    rz = rz_ref[0]
    r2 = r2_ref[0]
    cf = cf_ref[0]
    dst = dst_ref[0]

    rnorm = jnp.sqrt(r2)
    fc = 0.5 + 0.5 * jnp.cos(jnp.pi * rnorm / _R_CUTOFF)
    eid = e * _EB + lax.broadcasted_iota(jnp.int32, (1, _EB), 1)
    valid = (eid < _N_EDGES).astype(jnp.float32)
    s = fc * fc * cf * valid

    diff = rnorm - rs_ref[...]
    radial = jnp.exp(-_ALPHA * diff * diff)
    one = jnp.ones_like(rx)
    mono = jnp.concatenate(
        [one, rz, ry, rx, rz * rz, ry * rz, ry * ry, rx * rz, rx * ry,
         rx * rx], axis=0) * s
    dens = (radial[:, None, :] * mono[None, :, :]).reshape(_N_RS * _N_L, _EB)
    dens_b = dens.astype(jnp.bfloat16)

    node = n * _NC + lax.broadcasted_iota(jnp.int32, (_NC, _EB), 0)
    onehot = (node == dst).astype(jnp.bfloat16)

    acc_ref[...] += lax.dot_general(
        onehot, dens_b, (((1,), (1,)), ((), ())),
        preferred_element_type=jnp.float32)

    @pl.when(e == pl.num_programs(1) - 1)
    def _():
        a = acc_ref[...]
        sq = preidx_ref[...] * a * a
        out_ref[...] = lax.dot_general(
            sq, w_ref[...], (((1,), (0,)), ((), ())),
            preferred_element_type=jnp.float32)


def kernel(x, edge_index, atomic_numbers, coeffs):
    x = x.astype(jnp.float32)
    src = edge_index[0].astype(jnp.int32)
    dst = edge_index[1].astype(jnp.int32)
    an = atomic_numbers.astype(jnp.int32)
    cf_flat = coeffs.astype(jnp.float32).reshape(-1)

    pad = _E_PAD - _N_EDGES
    srcp = jnp.concatenate([src, jnp.zeros((pad,), jnp.int32)])
    dstp = jnp.concatenate([dst, jnp.zeros((pad,), jnp.int32)])

    rx, ry, rz, r2, cfe = _make_sc_edge()(
        x[:, 0], x[:, 1], x[:, 2], an, cf_flat, srcp, dstp)

    def blk(a):
        return a.reshape(_N_EB, 1, _EB)

    rs_col = jnp.asarray(_R_S).reshape(_N_RS, 1)
    preidx = jnp.asarray(_PREIDX_ROW)
    w_proj = jnp.asarray(_W_PROJ)

    edge_spec = pl.BlockSpec((1, 1, _EB), lambda n, e: (e, 0, 0))
    out = pl.pallas_call(
        _tc_dense_kernel,
        grid=(_N_NB, _N_EB),
        in_specs=[edge_spec] * 6 + [
            pl.BlockSpec((_N_RS, 1), lambda n, e: (0, 0)),
            pl.BlockSpec((1, _N_RS * _N_L), lambda n, e: (0, 0)),
            pl.BlockSpec((_N_RS * _N_L, _N_RS * (_L + 1)),
                         lambda n, e: (0, 0)),
        ],
        out_specs=pl.BlockSpec((_NC, _N_RS * (_L + 1)), lambda n, e: (n, 0)),
        out_shape=jax.ShapeDtypeStruct((_N_NODES, _N_RS * (_L + 1)),
                                       jnp.float32),
        scratch_shapes=[pltpu.VMEM((_NC, _N_RS * _N_L), jnp.float32)],
    )(blk(rx), blk(ry), blk(rz), blk(r2), blk(cfe), blk(dstp),
      rs_col, preidx, w_proj)
    return out


# speedup vs baseline: 2.1685x; 1.0434x over previous
import functools
import math

import numpy as np
import jax
import jax.numpy as jnp
from jax import lax
from jax.experimental import pallas as pl
from jax.experimental.pallas import tpu as pltpu
from jax.experimental.pallas import tpu_sc as plsc

_L = 2
_N_NODES = 10000
_N_EDGES = 160000
_MAX_NUMBER = 8
_R_CUTOFF = 6.0
_N_RS = 64
_ALPHA = 16.0

def _gen_l_list():
    rows = []
    for single_l in range(_L + 1):
        for ii in range(single_l + 1):
            for jj in range(single_l + 1):
                for kk in range(single_l + 1):
                    if ii + jj + kk == single_l:
                        rows.append([ii, jj, kk])
    return np.array(rows, dtype=np.int32)

_L_LIST = _gen_l_list()
_N_L = _L_LIST.shape[0]
_L_SUM = _L_LIST.sum(-1)
_fact = np.vectorize(math.factorial)
_L_PREIDX = (_fact(_L_SUM.astype(np.int64)) /
             _fact(_L_LIST.astype(np.int64)).prod(-1)).astype(np.float32)
_R_S = np.linspace(0.0, 6.0, _N_RS).astype(np.float32)

_PREIDX_ROW = np.tile(_L_PREIDX, _N_RS)[None, :]
_W_PROJ = np.zeros((_N_RS * _N_L, _N_RS * (_L + 1)), dtype=np.float32)
for _k in range(_N_RS):
    for _l in range(_N_L):
        _W_PROJ[_k * _N_L + _l, _k * (_L + 1) + _L_SUM[_l]] = 1.0

_E_PAD = 163840
_EB = 2048
_N_EB = _E_PAD // _EB
_NC = 2000
_N_NB = _N_NODES // _NC
_NW = 32
_EPW = _E_PAD // _NW


def _sc_edge_kernel(xx_hbm, xy_hbm, xz_hbm, an_hbm, cf_hbm, src_hbm, dst_hbm,
                    o_rx, o_ry, o_rz, o_r2, o_cf,
                    xx_v, xy_v, xz_v, an_v, cf_v, src_v, dst_v,
                    rx_v, ry_v, rz_v, r2_v, cfo_v):
    wid = lax.axis_index("s") * 2 + lax.axis_index("c")
    base = wid * _EPW
    pltpu.sync_copy(xx_hbm, xx_v)
    pltpu.sync_copy(xy_hbm, xy_v)
    pltpu.sync_copy(xz_hbm, xz_v)
    pltpu.sync_copy(an_hbm, an_v)
    pltpu.sync_copy(cf_hbm, cf_v)
    pltpu.sync_copy(src_hbm.at[pl.ds(base, _EPW)], src_v)
    pltpu.sync_copy(dst_hbm.at[pl.ds(base, _EPW)], dst_v)

    def body(i, _):
        off = i * 16
        s_idx = src_v[pl.ds(off, 16)]
        d_idx = dst_v[pl.ds(off, 16)]
        xs = plsc.load_gather(xx_v, [s_idx])
        ys = plsc.load_gather(xy_v, [s_idx])
        zs = plsc.load_gather(xz_v, [s_idx])
        xd = plsc.load_gather(xx_v, [d_idx])
        yd = plsc.load_gather(xy_v, [d_idx])
        zd = plsc.load_gather(xz_v, [d_idx])
        rx = (xd - xs) + 1e-6
        ry = (yd - ys) + 1e-6
        rz = (zd - zs) + 1e-6
        r2 = rx * rx + ry * ry + rz * rz
        zsrc = plsc.load_gather(an_v, [s_idx])
        zdst = plsc.load_gather(an_v, [d_idx])
        zmin = jnp.minimum(zsrc, zdst)
        zmax = jnp.maximum(zsrc, zdst)
        cf = plsc.load_gather(cf_v, [zmin * _MAX_NUMBER + zmax])
        rx_v[pl.ds(off, 16)] = rx
        ry_v[pl.ds(off, 16)] = ry
        rz_v[pl.ds(off, 16)] = rz
        r2_v[pl.ds(off, 16)] = r2
        cfo_v[pl.ds(off, 16)] = cf
        return 0

    lax.fori_loop(0, _EPW // 16, body, 0)

    pltpu.sync_copy(rx_v, o_rx.at[pl.ds(base, _EPW)])
    pltpu.sync_copy(ry_v, o_ry.at[pl.ds(base, _EPW)])
    pltpu.sync_copy(rz_v, o_rz.at[pl.ds(base, _EPW)])
    pltpu.sync_copy(r2_v, o_r2.at[pl.ds(base, _EPW)])
    pltpu.sync_copy(cfo_v, o_cf.at[pl.ds(base, _EPW)])


def _make_sc_edge():
    return functools.partial(
        pl.kernel,
        out_type=[jax.ShapeDtypeStruct((_E_PAD,), jnp.float32)] * 5,
        mesh=plsc.VectorSubcoreMesh(core_axis_name="c", subcore_axis_name="s"),
        compiler_params=pltpu.CompilerParams(needs_layout_passes=False),
        scratch_types=[
        pltpu.VMEM((_N_NODES,), jnp.float32),
        pltpu.VMEM((_N_NODES,), jnp.float32),
        pltpu.VMEM((_N_NODES,), jnp.float32),
        pltpu.VMEM((_N_NODES,), jnp.int32),
        pltpu.VMEM((_MAX_NUMBER * _MAX_NUMBER,), jnp.float32),
        pltpu.VMEM((_EPW,), jnp.int32),
        pltpu.VMEM((_EPW,), jnp.int32),
        pltpu.VMEM((_EPW,), jnp.float32),
        pltpu.VMEM((_EPW,), jnp.float32),
        pltpu.VMEM((_EPW,), jnp.float32),
            pltpu.VMEM((_EPW,), jnp.float32),
            pltpu.VMEM((_EPW,), jnp.float32),
        ],
    )(_sc_edge_kernel)


def _tc_dense_kernel(rx_ref, ry_ref, rz_ref, r2_ref, cf_ref, dst_ref,
                     rs_ref, preidx_ref, w_ref, out_ref, acc_ref):
    n = pl.program_id(0)
    e = pl.program_id(1)

    @pl.when(e == 0)
    def _():
        acc_ref[...] = jnp.zeros_like(acc_ref)

    rx = rx_ref[0]
    ry = ry_ref[0]
    rz = rz_ref[0]
    r2 = r2_ref[0]
    cf = cf_ref[0]
    dst = dst_ref[0]

    rnorm = jnp.sqrt(r2)
    fc = 0.5 + 0.5 * jnp.cos(jnp.pi * rnorm / _R_CUTOFF)
    eid = e * _EB + lax.broadcasted_iota(jnp.int32, (1, _EB), 1)
    valid = (eid < _N_EDGES).astype(jnp.float32)
    s = fc * fc * cf * valid

    diff = rnorm - rs_ref[...]
    radial = jnp.exp(-_ALPHA * diff * diff)
    one = jnp.ones_like(rx)
    mono = jnp.concatenate(
        [one, rz, ry, rx, rz * rz, ry * rz, ry * ry, rx * rz, rx * ry,
         rx * rx], axis=0) * s
    dens = (radial[:, None, :] * mono[None, :, :]).reshape(_N_RS * _N_L, _EB)
    dens_b = dens.astype(jnp.bfloat16)

    node = n * _NC + lax.broadcasted_iota(jnp.int32, (_NC, _EB), 0)
    onehot = (node == dst).astype(jnp.bfloat16)

    acc_ref[...] += lax.dot_general(
        onehot, dens_b, (((1,), (1,)), ((), ())),
        preferred_element_type=jnp.float32)

    @pl.when(e == pl.num_programs(1) - 1)
    def _():
        a = acc_ref[...]
        sq = preidx_ref[...] * a * a
        out_ref[...] = lax.dot_general(
            sq, w_ref[...], (((1,), (0,)), ((), ())),
            preferred_element_type=jnp.float32)


def kernel(x, edge_index, atomic_numbers, coeffs):
    x = x.astype(jnp.float32)
    src = edge_index[0].astype(jnp.int32)
    dst = edge_index[1].astype(jnp.int32)
    an = atomic_numbers.astype(jnp.int32)
    cf_flat = coeffs.astype(jnp.float32).reshape(-1)

    pad = _E_PAD - _N_EDGES
    srcp = jnp.concatenate([src, jnp.zeros((pad,), jnp.int32)])
    dstp = jnp.concatenate([dst, jnp.zeros((pad,), jnp.int32)])

    rx, ry, rz, r2, cfe = _make_sc_edge()(
        x[:, 0], x[:, 1], x[:, 2], an, cf_flat, srcp, dstp)

    def blk(a):
        return a.reshape(_N_EB, 1, _EB)

    rs_col = jnp.asarray(_R_S).reshape(_N_RS, 1)
    preidx = jnp.asarray(_PREIDX_ROW)
    w_proj = jnp.asarray(_W_PROJ)

    edge_spec = pl.BlockSpec((1, 1, _EB), lambda n, e: (e, 0, 0))
    out = pl.pallas_call(
        _tc_dense_kernel,
        grid=(_N_NB, _N_EB),
        in_specs=[edge_spec] * 6 + [
            pl.BlockSpec((_N_RS, 1), lambda n, e: (0, 0)),
            pl.BlockSpec((1, _N_RS * _N_L), lambda n, e: (0, 0)),
            pl.BlockSpec((_N_RS * _N_L, _N_RS * (_L + 1)),
                         lambda n, e: (0, 0)),
        ],
        out_specs=pl.BlockSpec((_NC, _N_RS * (_L + 1)), lambda n, e: (n, 0)),
        out_shape=jax.ShapeDtypeStruct((_N_NODES, _N_RS * (_L + 1)),
                                       jnp.float32),
        scratch_shapes=[pltpu.VMEM((_NC, _N_RS * _N_L), jnp.float32)],
    )(blk(rx), blk(ry), blk(rz), blk(r2), blk(cfe), blk(dstp),
      rs_col, preidx, w_proj)
    return out


# speedup vs baseline: 2.2411x; 1.0335x over previous
import functools
import math

import numpy as np
import jax
import jax.numpy as jnp
from jax import lax
from jax.experimental import pallas as pl
from jax.experimental.pallas import tpu as pltpu
from jax.experimental.pallas import tpu_sc as plsc

_L = 2
_N_NODES = 10000
_N_EDGES = 160000
_MAX_NUMBER = 8
_R_CUTOFF = 6.0
_N_RS = 64
_ALPHA = 16.0

def _gen_l_list():
    rows = []
    for single_l in range(_L + 1):
        for ii in range(single_l + 1):
            for jj in range(single_l + 1):
                for kk in range(single_l + 1):
                    if ii + jj + kk == single_l:
                        rows.append([ii, jj, kk])
    return np.array(rows, dtype=np.int32)

_L_LIST = _gen_l_list()
_N_L = _L_LIST.shape[0]
_L_SUM = _L_LIST.sum(-1)
_fact = np.vectorize(math.factorial)
_L_PREIDX = (_fact(_L_SUM.astype(np.int64)) /
             _fact(_L_LIST.astype(np.int64)).prod(-1)).astype(np.float32)
_R_S = np.linspace(0.0, 6.0, _N_RS).astype(np.float32)

_PREIDX_ROW = np.tile(_L_PREIDX, _N_RS)[None, :]
_W_PROJ = np.zeros((_N_RS * _N_L, _N_RS * (_L + 1)), dtype=np.float32)
for _k in range(_N_RS):
    for _l in range(_N_L):
        _W_PROJ[_k * _N_L + _l, _k * (_L + 1) + _L_SUM[_l]] = 1.0

_E_PAD = 163840
_EB = 4096
_N_EB = _E_PAD // _EB
_NC = 2000
_N_NB = _N_NODES // _NC
_NW = 32
_EPW = _E_PAD // _NW


def _sc_edge_kernel(xx_hbm, xy_hbm, xz_hbm, an_hbm, cf_hbm, src_hbm, dst_hbm,
                    o_rx, o_ry, o_rz, o_r2, o_cf,
                    xx_v, xy_v, xz_v, an_v, cf_v, src_v, dst_v,
                    rx_v, ry_v, rz_v, r2_v, cfo_v):
    wid = lax.axis_index("s") * 2 + lax.axis_index("c")
    base = wid * _EPW
    pltpu.sync_copy(xx_hbm, xx_v)
    pltpu.sync_copy(xy_hbm, xy_v)
    pltpu.sync_copy(xz_hbm, xz_v)
    pltpu.sync_copy(an_hbm, an_v)
    pltpu.sync_copy(cf_hbm, cf_v)
    pltpu.sync_copy(src_hbm.at[pl.ds(base, _EPW)], src_v)
    pltpu.sync_copy(dst_hbm.at[pl.ds(base, _EPW)], dst_v)

    def body(i, _):
        off = i * 16
        s_idx = src_v[pl.ds(off, 16)]
        d_idx = dst_v[pl.ds(off, 16)]
        xs = plsc.load_gather(xx_v, [s_idx])
        ys = plsc.load_gather(xy_v, [s_idx])
        zs = plsc.load_gather(xz_v, [s_idx])
        xd = plsc.load_gather(xx_v, [d_idx])
        yd = plsc.load_gather(xy_v, [d_idx])
        zd = plsc.load_gather(xz_v, [d_idx])
        rx = (xd - xs) + 1e-6
        ry = (yd - ys) + 1e-6
        rz = (zd - zs) + 1e-6
        r2 = rx * rx + ry * ry + rz * rz
        zsrc = plsc.load_gather(an_v, [s_idx])
        zdst = plsc.load_gather(an_v, [d_idx])
        zmin = jnp.minimum(zsrc, zdst)
        zmax = jnp.maximum(zsrc, zdst)
        cf = plsc.load_gather(cf_v, [zmin * _MAX_NUMBER + zmax])
        rx_v[pl.ds(off, 16)] = rx
        ry_v[pl.ds(off, 16)] = ry
        rz_v[pl.ds(off, 16)] = rz
        r2_v[pl.ds(off, 16)] = r2
        cfo_v[pl.ds(off, 16)] = cf
        return 0

    lax.fori_loop(0, _EPW // 16, body, 0)

    pltpu.sync_copy(rx_v, o_rx.at[pl.ds(base, _EPW)])
    pltpu.sync_copy(ry_v, o_ry.at[pl.ds(base, _EPW)])
    pltpu.sync_copy(rz_v, o_rz.at[pl.ds(base, _EPW)])
    pltpu.sync_copy(r2_v, o_r2.at[pl.ds(base, _EPW)])
    pltpu.sync_copy(cfo_v, o_cf.at[pl.ds(base, _EPW)])


def _make_sc_edge():
    return functools.partial(
        pl.kernel,
        out_type=[jax.ShapeDtypeStruct((_E_PAD,), jnp.float32)] * 5,
        mesh=plsc.VectorSubcoreMesh(core_axis_name="c", subcore_axis_name="s"),
        compiler_params=pltpu.CompilerParams(needs_layout_passes=False),
        scratch_types=[
        pltpu.VMEM((_N_NODES,), jnp.float32),
        pltpu.VMEM((_N_NODES,), jnp.float32),
        pltpu.VMEM((_N_NODES,), jnp.float32),
        pltpu.VMEM((_N_NODES,), jnp.int32),
        pltpu.VMEM((_MAX_NUMBER * _MAX_NUMBER,), jnp.float32),
        pltpu.VMEM((_EPW,), jnp.int32),
        pltpu.VMEM((_EPW,), jnp.int32),
        pltpu.VMEM((_EPW,), jnp.float32),
        pltpu.VMEM((_EPW,), jnp.float32),
        pltpu.VMEM((_EPW,), jnp.float32),
            pltpu.VMEM((_EPW,), jnp.float32),
            pltpu.VMEM((_EPW,), jnp.float32),
        ],
    )(_sc_edge_kernel)


def _tc_dense_kernel(rx_ref, ry_ref, rz_ref, r2_ref, cf_ref, dst_ref,
                     rs_ref, preidx_ref, w_ref, out_ref, acc_ref):
    n = pl.program_id(0)
    e = pl.program_id(1)

    @pl.when(e == 0)
    def _():
        acc_ref[...] = jnp.zeros_like(acc_ref)

    rx = rx_ref[0]
    ry = ry_ref[0]
    rz = rz_ref[0]
    r2 = r2_ref[0]
    cf = cf_ref[0]
    dst = dst_ref[0]

    rnorm = jnp.sqrt(r2)
    fc = 0.5 + 0.5 * jnp.cos(jnp.pi * rnorm / _R_CUTOFF)
    eid = e * _EB + lax.broadcasted_iota(jnp.int32, (1, _EB), 1)
    valid = (eid < _N_EDGES).astype(jnp.float32)
    s = fc * fc * cf * valid

    diff = rnorm - rs_ref[...]
    radial = jnp.exp(-_ALPHA * diff * diff)
    one = jnp.ones_like(rx)
    mono = jnp.concatenate(
        [one, rz, ry, rx, rz * rz, ry * rz, ry * ry, rx * rz, rx * ry,
         rx * rx], axis=0) * s
    dens = (radial[:, None, :] * mono[None, :, :]).reshape(_N_RS * _N_L, _EB)
    dens_b = dens.astype(jnp.bfloat16)

    node = n * _NC + lax.broadcasted_iota(jnp.int32, (_NC, _EB), 0)
    onehot = (node == dst).astype(jnp.bfloat16)

    acc_ref[...] += lax.dot_general(
        onehot, dens_b, (((1,), (1,)), ((), ())),
        preferred_element_type=jnp.float32)

    @pl.when(e == pl.num_programs(1) - 1)
    def _():
        a = acc_ref[...]
        sq = preidx_ref[...] * a * a
        out_ref[...] = lax.dot_general(
            sq, w_ref[...], (((1,), (0,)), ((), ())),
            preferred_element_type=jnp.float32)


def kernel(x, edge_index, atomic_numbers, coeffs):
    x = x.astype(jnp.float32)
    src = edge_index[0].astype(jnp.int32)
    dst = edge_index[1].astype(jnp.int32)
    an = atomic_numbers.astype(jnp.int32)
    cf_flat = coeffs.astype(jnp.float32).reshape(-1)

    pad = _E_PAD - _N_EDGES
    srcp = jnp.concatenate([src, jnp.zeros((pad,), jnp.int32)])
    dstp = jnp.concatenate([dst, jnp.zeros((pad,), jnp.int32)])

    rx, ry, rz, r2, cfe = _make_sc_edge()(
        x[:, 0], x[:, 1], x[:, 2], an, cf_flat, srcp, dstp)

    def blk(a):
        return a.reshape(_N_EB, 1, _EB)

    rs_col = jnp.asarray(_R_S).reshape(_N_RS, 1)
    preidx = jnp.asarray(_PREIDX_ROW)
    w_proj = jnp.asarray(_W_PROJ)

    edge_spec = pl.BlockSpec((1, 1, _EB), lambda n, e: (e, 0, 0))
    out = pl.pallas_call(
        _tc_dense_kernel,
        grid=(_N_NB, _N_EB),
        in_specs=[edge_spec] * 6 + [
            pl.BlockSpec((_N_RS, 1), lambda n, e: (0, 0)),
            pl.BlockSpec((1, _N_RS * _N_L), lambda n, e: (0, 0)),
            pl.BlockSpec((_N_RS * _N_L, _N_RS * (_L + 1)),
                         lambda n, e: (0, 0)),
        ],
        out_specs=pl.BlockSpec((_NC, _N_RS * (_L + 1)), lambda n, e: (n, 0)),
        out_shape=jax.ShapeDtypeStruct((_N_NODES, _N_RS * (_L + 1)),
                                       jnp.float32),
        scratch_shapes=[pltpu.VMEM((_NC, _N_RS * _N_L), jnp.float32)],
    )(blk(rx), blk(ry), blk(rz), blk(r2), blk(cfe), blk(dstp),
      rs_col, preidx, w_proj)
    return out


# speedup vs baseline: 2.2924x; 1.0229x over previous
import functools
import math

import numpy as np
import jax
import jax.numpy as jnp
from jax import lax
from jax.experimental import pallas as pl
from jax.experimental.pallas import tpu as pltpu
from jax.experimental.pallas import tpu_sc as plsc

_L = 2
_N_NODES = 10000
_N_EDGES = 160000
_MAX_NUMBER = 8
_R_CUTOFF = 6.0
_N_RS = 64
_ALPHA = 16.0

def _gen_l_list():
    rows = []
    for single_l in range(_L + 1):
        for ii in range(single_l + 1):
            for jj in range(single_l + 1):
                for kk in range(single_l + 1):
                    if ii + jj + kk == single_l:
                        rows.append([ii, jj, kk])
    return np.array(rows, dtype=np.int32)

_L_LIST = _gen_l_list()
_N_L = _L_LIST.shape[0]
_L_SUM = _L_LIST.sum(-1)
_fact = np.vectorize(math.factorial)
_L_PREIDX = (_fact(_L_SUM.astype(np.int64)) /
             _fact(_L_LIST.astype(np.int64)).prod(-1)).astype(np.float32)
_R_S = np.linspace(0.0, 6.0, _N_RS).astype(np.float32)

_PREIDX_ROW = np.tile(_L_PREIDX, _N_RS)[None, :]
_W_PROJ = np.zeros((_N_RS * _N_L, _N_RS * (_L + 1)), dtype=np.float32)
for _k in range(_N_RS):
    for _l in range(_N_L):
        _W_PROJ[_k * _N_L + _l, _k * (_L + 1) + _L_SUM[_l]] = 1.0

_E_PAD = 163840
_EB = 8192
_N_EB = _E_PAD // _EB
_NC = 2000
_N_NB = _N_NODES // _NC
_NW = 32
_EPW = _E_PAD // _NW


def _sc_edge_kernel(xx_hbm, xy_hbm, xz_hbm, an_hbm, cf_hbm, src_hbm, dst_hbm,
                    o_rx, o_ry, o_rz, o_r2, o_cf,
                    xx_v, xy_v, xz_v, an_v, cf_v, src_v, dst_v,
                    rx_v, ry_v, rz_v, r2_v, cfo_v):
    wid = lax.axis_index("s") * 2 + lax.axis_index("c")
    base = wid * _EPW
    pltpu.sync_copy(xx_hbm, xx_v)
    pltpu.sync_copy(xy_hbm, xy_v)
    pltpu.sync_copy(xz_hbm, xz_v)
    pltpu.sync_copy(an_hbm, an_v)
    pltpu.sync_copy(cf_hbm, cf_v)
    pltpu.sync_copy(src_hbm.at[pl.ds(base, _EPW)], src_v)
    pltpu.sync_copy(dst_hbm.at[pl.ds(base, _EPW)], dst_v)

    def body(i, _):
        off = i * 16
        s_idx = src_v[pl.ds(off, 16)]
        d_idx = dst_v[pl.ds(off, 16)]
        xs = plsc.load_gather(xx_v, [s_idx])
        ys = plsc.load_gather(xy_v, [s_idx])
        zs = plsc.load_gather(xz_v, [s_idx])
        xd = plsc.load_gather(xx_v, [d_idx])
        yd = plsc.load_gather(xy_v, [d_idx])
        zd = plsc.load_gather(xz_v, [d_idx])
        rx = (xd - xs) + 1e-6
        ry = (yd - ys) + 1e-6
        rz = (zd - zs) + 1e-6
        r2 = rx * rx + ry * ry + rz * rz
        zsrc = plsc.load_gather(an_v, [s_idx])
        zdst = plsc.load_gather(an_v, [d_idx])
        zmin = jnp.minimum(zsrc, zdst)
        zmax = jnp.maximum(zsrc, zdst)
        cf = plsc.load_gather(cf_v, [zmin * _MAX_NUMBER + zmax])
        rx_v[pl.ds(off, 16)] = rx
        ry_v[pl.ds(off, 16)] = ry
        rz_v[pl.ds(off, 16)] = rz
        r2_v[pl.ds(off, 16)] = r2
        cfo_v[pl.ds(off, 16)] = cf
        return 0

    lax.fori_loop(0, _EPW // 16, body, 0)

    pltpu.sync_copy(rx_v, o_rx.at[pl.ds(base, _EPW)])
    pltpu.sync_copy(ry_v, o_ry.at[pl.ds(base, _EPW)])
    pltpu.sync_copy(rz_v, o_rz.at[pl.ds(base, _EPW)])
    pltpu.sync_copy(r2_v, o_r2.at[pl.ds(base, _EPW)])
    pltpu.sync_copy(cfo_v, o_cf.at[pl.ds(base, _EPW)])


def _make_sc_edge():
    return functools.partial(
        pl.kernel,
        out_type=[jax.ShapeDtypeStruct((_E_PAD,), jnp.float32)] * 5,
        mesh=plsc.VectorSubcoreMesh(core_axis_name="c", subcore_axis_name="s"),
        compiler_params=pltpu.CompilerParams(needs_layout_passes=False),
        scratch_types=[
        pltpu.VMEM((_N_NODES,), jnp.float32),
        pltpu.VMEM((_N_NODES,), jnp.float32),
        pltpu.VMEM((_N_NODES,), jnp.float32),
        pltpu.VMEM((_N_NODES,), jnp.int32),
        pltpu.VMEM((_MAX_NUMBER * _MAX_NUMBER,), jnp.float32),
        pltpu.VMEM((_EPW,), jnp.int32),
        pltpu.VMEM((_EPW,), jnp.int32),
        pltpu.VMEM((_EPW,), jnp.float32),
        pltpu.VMEM((_EPW,), jnp.float32),
        pltpu.VMEM((_EPW,), jnp.float32),
            pltpu.VMEM((_EPW,), jnp.float32),
            pltpu.VMEM((_EPW,), jnp.float32),
        ],
    )(_sc_edge_kernel)


def _tc_dense_kernel(rx_ref, ry_ref, rz_ref, r2_ref, cf_ref, dst_ref,
                     rs_ref, preidx_ref, w_ref, out_ref, acc_ref):
    n = pl.program_id(0)
    e = pl.program_id(1)

    @pl.when(e == 0)
    def _():
        acc_ref[...] = jnp.zeros_like(acc_ref)

    rx = rx_ref[0]
    ry = ry_ref[0]
    rz = rz_ref[0]
    r2 = r2_ref[0]
    cf = cf_ref[0]
    dst = dst_ref[0]

    rnorm = jnp.sqrt(r2)
    fc = 0.5 + 0.5 * jnp.cos(jnp.pi * rnorm / _R_CUTOFF)
    eid = e * _EB + lax.broadcasted_iota(jnp.int32, (1, _EB), 1)
    valid = (eid < _N_EDGES).astype(jnp.float32)
    s = fc * fc * cf * valid

    diff = rnorm - rs_ref[...]
    radial = jnp.exp(-_ALPHA * diff * diff)
    one = jnp.ones_like(rx)
    mono = jnp.concatenate(
        [one, rz, ry, rx, rz * rz, ry * rz, ry * ry, rx * rz, rx * ry,
         rx * rx], axis=0) * s
    dens = (radial[:, None, :] * mono[None, :, :]).reshape(_N_RS * _N_L, _EB)
    dens_b = dens.astype(jnp.bfloat16)

    node = n * _NC + lax.broadcasted_iota(jnp.int32, (_NC, _EB), 0)
    onehot = (node == dst).astype(jnp.bfloat16)

    acc_ref[...] += lax.dot_general(
        onehot, dens_b, (((1,), (1,)), ((), ())),
        preferred_element_type=jnp.float32)

    @pl.when(e == pl.num_programs(1) - 1)
    def _():
        a = acc_ref[...]
        sq = preidx_ref[...] * a * a
        out_ref[...] = lax.dot_general(
            sq, w_ref[...], (((1,), (0,)), ((), ())),
            preferred_element_type=jnp.float32)


def kernel(x, edge_index, atomic_numbers, coeffs):
    x = x.astype(jnp.float32)
    src = edge_index[0].astype(jnp.int32)
    dst = edge_index[1].astype(jnp.int32)
    an = atomic_numbers.astype(jnp.int32)
    cf_flat = coeffs.astype(jnp.float32).reshape(-1)

    pad = _E_PAD - _N_EDGES
    srcp = jnp.concatenate([src, jnp.zeros((pad,), jnp.int32)])
    dstp = jnp.concatenate([dst, jnp.zeros((pad,), jnp.int32)])

    rx, ry, rz, r2, cfe = _make_sc_edge()(
        x[:, 0], x[:, 1], x[:, 2], an, cf_flat, srcp, dstp)

    def blk(a):
        return a.reshape(_N_EB, 1, _EB)

    rs_col = jnp.asarray(_R_S).reshape(_N_RS, 1)
    preidx = jnp.asarray(_PREIDX_ROW)
    w_proj = jnp.asarray(_W_PROJ)

    edge_spec = pl.BlockSpec((1, 1, _EB), lambda n, e: (e, 0, 0))
    out = pl.pallas_call(
        _tc_dense_kernel,
        grid=(_N_NB, _N_EB),
        in_specs=[edge_spec] * 6 + [
            pl.BlockSpec((_N_RS, 1), lambda n, e: (0, 0)),
            pl.BlockSpec((1, _N_RS * _N_L), lambda n, e: (0, 0)),
            pl.BlockSpec((_N_RS * _N_L, _N_RS * (_L + 1)),
                         lambda n, e: (0, 0)),
        ],
        out_specs=pl.BlockSpec((_NC, _N_RS * (_L + 1)), lambda n, e: (n, 0)),
        out_shape=jax.ShapeDtypeStruct((_N_NODES, _N_RS * (_L + 1)),
                                       jnp.float32),
        scratch_shapes=[pltpu.VMEM((_NC, _N_RS * _N_L), jnp.float32)],
        compiler_params=pltpu.CompilerParams(
            vmem_limit_bytes=120 * 1024 * 1024),
    )(blk(rx), blk(ry), blk(rz), blk(r2), blk(cfe), blk(dstp),
      rs_col, preidx, w_proj)
    return out
